# baseline (speedup 1.0000x reference)
import jax
import jax.numpy as jnp
from jax import lax
from jax.experimental import pallas as pl
from jax.experimental.pallas import tpu as pltpu

N = 2048
DIM = 768
E = 8
FF = DIM * 4
THRESHOLD = 0.3

BTG = 256
NIG = N // BTG

BTC = 384
NJ = -(-N // BTC)
FFB = 768
NK = FF // FFB


def _gate_kernel(x_ref, wg_ref, bg_ref, scores_ref, w_ref, pos_ref, cnt_ref,
                 carry_ref):
    i = pl.program_id(0)

    @pl.when(i == 0)
    def _init():
        carry_ref[...] = jnp.zeros((E, 1), jnp.float32)

    logits = lax.dot_general(wg_ref[...], x_ref[...],
                             (((0,), (1,)), ((), ())),
                             preferred_element_type=jnp.float32)
    logits = logits + bg_ref[...]
    mx0 = jnp.max(logits, axis=0, keepdims=True)
    ex = jnp.exp(logits - mx0)
    scores = ex / jnp.sum(ex, axis=0, keepdims=True)
    scores_ref[...] = scores

    mask = (scores >= THRESHOLD).astype(jnp.float32)
    masked = scores * mask
    denom_raw = jnp.sum(masked, axis=0, keepdims=True)
    normed = masked / (denom_raw + 1e-6)
    iot = lax.broadcasted_iota(jnp.int32, scores.shape, 0)
    mxs = jnp.max(scores, axis=0, keepdims=True)
    cand = jnp.where(scores == mxs, iot, E)
    top1 = jnp.min(cand, axis=0, keepdims=True)
    onehot = (iot == top1).astype(jnp.float32)
    w = jnp.where(denom_raw == 0.0, onehot, normed)
    w_ref[...] = w

    act = (w > 0.0).astype(jnp.float32)
    rowi = lax.broadcasted_iota(jnp.int32, (BTG, BTG), 0)
    coli = lax.broadcasted_iota(jnp.int32, (BTG, BTG), 1)
    tri = (rowi < coli).astype(jnp.float32)
    pos = lax.dot_general(act, tri, (((1,), (0,)), ((), ())),
                          preferred_element_type=jnp.float32)
    pos = pos + carry_ref[...]
    pos_ref[...] = pos.astype(jnp.int32)
    new_carry = carry_ref[...] + jnp.sum(act, axis=1, keepdims=True)
    carry_ref[...] = new_carry

    @pl.when(i == NIG - 1)
    def _fin():
        cnt_ref[...] = new_carry.astype(jnp.int32)


def _gate(x, Wg, bg):
    return pl.pallas_call(
        _gate_kernel,
        grid=(NIG,),
        in_specs=[
            pl.BlockSpec((BTG, DIM), lambda i: (i, 0)),
            pl.BlockSpec((DIM, E), lambda i: (0, 0)),
            pl.BlockSpec((E, 1), lambda i: (0, 0)),
        ],
        out_specs=(
            pl.BlockSpec((E, BTG), lambda i: (0, i)),
            pl.BlockSpec((E, BTG), lambda i: (0, i)),
            pl.BlockSpec((E, BTG), lambda i: (0, i)),
            pl.BlockSpec((E, 1), lambda i: (0, 0)),
        ),
        out_shape=(
            jax.ShapeDtypeStruct((E, N), jnp.float32),
            jax.ShapeDtypeStruct((E, N), jnp.float32),
            jax.ShapeDtypeStruct((E, N), jnp.int32),
            jax.ShapeDtypeStruct((E, 1), jnp.int32),
        ),
        scratch_shapes=[pltpu.VMEM((E, 1), jnp.float32)],
    )(x, Wg, bg.reshape(E, 1))


def _expert_kernel(cnt_ref, einp_ref, w1_ref, b1_ref, w2_ref, b2_ref,
                   wT_ref, posT_ref, out_ref, xg_ref, o_ref):
    e = pl.program_id(0)
    k = pl.program_id(1)

    @pl.when((e == 0) & (k == 0))
    def _init():
        out_ref[...] = jnp.zeros((N, DIM), jnp.float32)

    nblk = lax.div(cnt_ref[e] + BTC - 1, BTC)
    pos = posT_ref[pl.ds(e, 1), :]
    w = wT_ref[pl.ds(e, 1), :]
    act = w > 0.0

    @pl.when(k == 0)
    def _gather():
        def gbody(j, c):
            rr = lax.broadcasted_iota(jnp.int32, (BTC, N), 0) + j * BTC
            pmat = ((pos == rr) & act).astype(jnp.bfloat16)
            xg_ref[pl.ds(j * BTC, BTC), :] = lax.dot_general(
                pmat, einp_ref[0].astype(jnp.bfloat16),
                (((1,), (0,)), ((), ())),
                preferred_element_type=jnp.float32)
            return c

        lax.fori_loop(0, nblk, gbody, 0)

    def fbody(j, c):
        rows = pl.ds(j * BTC, BTC)
        h = lax.dot_general(xg_ref[rows, :].astype(jnp.bfloat16),
                            w1_ref[0].astype(jnp.bfloat16),
                            (((1,), (0,)), ((), ())),
                            preferred_element_type=jnp.float32)
        h = jax.nn.gelu(h + b1_ref[0, 0])
        contrib = lax.dot_general(h.astype(jnp.bfloat16),
                                  w2_ref[0].astype(jnp.bfloat16),
                                  (((1,), (0,)), ((), ())),
                                  preferred_element_type=jnp.float32)
        prev = jnp.where(k == 0, 0.0, o_ref[rows, :])
        total = prev + contrib
        o_ref[rows, :] = total

        @pl.when(k == NK - 1)
        def _fuse():
            rr = lax.broadcasted_iota(jnp.int32, (BTC, N), 0) + j * BTC
            gmat = jnp.where((pos == rr) & act, w, 0.0)
            opb = total + b2_ref[pl.ds(e, 1), :]
            out_ref[...] += lax.dot_general(
                gmat.astype(jnp.bfloat16), opb.astype(jnp.bfloat16),
                (((0,), (0,)), ((), ())),
                preferred_element_type=jnp.float32)

        return c

    lax.fori_loop(0, nblk, fbody, 0)


def _experts(counts, expert_inputs, W1, b1, W2, b2, wT, posT):
    grid_spec = pltpu.PrefetchScalarGridSpec(
        num_scalar_prefetch=1,
        grid=(E, NK),
        in_specs=[
            pl.BlockSpec((1, N, DIM), lambda e, k, cnt: (e, 0, 0)),
            pl.BlockSpec((1, DIM, FFB), lambda e, k, cnt: (e, 0, k)),
            pl.BlockSpec((1, 1, FFB), lambda e, k, cnt: (e, 0, k)),
            pl.BlockSpec((1, FFB, DIM), lambda e, k, cnt: (e, k, 0)),
            pl.BlockSpec((E, DIM), lambda e, k, cnt: (0, 0)),
            pl.BlockSpec((E, N), lambda e, k, cnt: (0, 0)),
            pl.BlockSpec((E, N), lambda e, k, cnt: (0, 0)),
        ],
        out_specs=pl.BlockSpec((N, DIM), lambda e, k, cnt: (0, 0)),
        scratch_shapes=[
            pltpu.VMEM((NJ * BTC, DIM), jnp.float32),
            pltpu.VMEM((NJ * BTC, DIM), jnp.float32),
        ],
    )
    return pl.pallas_call(
        _expert_kernel,
        grid_spec=grid_spec,
        out_shape=jax.ShapeDtypeStruct((N, DIM), jnp.float32),
    )(counts, expert_inputs, W1, b1.reshape(E, 1, FF), W2, b2, wT, posT)


def kernel(x, expert_inputs, Wg, bg, W1, b1, W2, b2):
    scoresT, wT, posT, counts = _gate(x, Wg, bg)
    fused = _experts(counts.reshape(E), expert_inputs, W1, b1, W2, b2,
                     wT, posT)
    return (fused, scoresT.T)


# speedup vs baseline: 1.0727x; 1.0727x over previous
import jax
import jax.numpy as jnp
from jax import lax
from jax.experimental import pallas as pl
from jax.experimental.pallas import tpu as pltpu

N = 2048
DIM = 768
E = 8
FF = DIM * 4
THRESHOLD = 0.3

BTG = 256
NIG = N // BTG

BTC = 320
NJ = -(-N // BTC)
FFB = 768
NK = FF // FFB


def _gate_kernel(x_ref, wg_ref, bg_ref, scores_ref, w_ref, pos_ref, cnt_ref,
                 carry_ref):
    i = pl.program_id(0)

    @pl.when(i == 0)
    def _init():
        carry_ref[...] = jnp.zeros((E, 1), jnp.float32)

    logits = lax.dot_general(wg_ref[...], x_ref[...],
                             (((0,), (1,)), ((), ())),
                             preferred_element_type=jnp.float32)
    logits = logits + bg_ref[...]
    mx0 = jnp.max(logits, axis=0, keepdims=True)
    ex = jnp.exp(logits - mx0)
    scores = ex / jnp.sum(ex, axis=0, keepdims=True)
    scores_ref[...] = scores

    mask = (scores >= THRESHOLD).astype(jnp.float32)
    masked = scores * mask
    denom_raw = jnp.sum(masked, axis=0, keepdims=True)
    normed = masked / (denom_raw + 1e-6)
    iot = lax.broadcasted_iota(jnp.int32, scores.shape, 0)
    mxs = jnp.max(scores, axis=0, keepdims=True)
    cand = jnp.where(scores == mxs, iot, E)
    top1 = jnp.min(cand, axis=0, keepdims=True)
    onehot = (iot == top1).astype(jnp.float32)
    w = jnp.where(denom_raw == 0.0, onehot, normed)
    w_ref[...] = w

    act = (w > 0.0).astype(jnp.float32)
    rowi = lax.broadcasted_iota(jnp.int32, (BTG, BTG), 0)
    coli = lax.broadcasted_iota(jnp.int32, (BTG, BTG), 1)
    tri = (rowi < coli).astype(jnp.float32)
    pos = lax.dot_general(act, tri, (((1,), (0,)), ((), ())),
                          preferred_element_type=jnp.float32)
    pos = pos + carry_ref[...]
    pos_ref[...] = pos.astype(jnp.int32)
    new_carry = carry_ref[...] + jnp.sum(act, axis=1, keepdims=True)
    carry_ref[...] = new_carry

    @pl.when(i == NIG - 1)
    def _fin():
        cnt_ref[...] = new_carry.astype(jnp.int32)


def _gate(x, Wg, bg):
    return pl.pallas_call(
        _gate_kernel,
        grid=(NIG,),
        in_specs=[
            pl.BlockSpec((BTG, DIM), lambda i: (i, 0)),
            pl.BlockSpec((DIM, E), lambda i: (0, 0)),
            pl.BlockSpec((E, 1), lambda i: (0, 0)),
        ],
        out_specs=(
            pl.BlockSpec((E, BTG), lambda i: (0, i)),
            pl.BlockSpec((E, BTG), lambda i: (0, i)),
            pl.BlockSpec((E, BTG), lambda i: (0, i)),
            pl.BlockSpec((E, 1), lambda i: (0, 0)),
        ),
        out_shape=(
            jax.ShapeDtypeStruct((E, N), jnp.float32),
            jax.ShapeDtypeStruct((E, N), jnp.float32),
            jax.ShapeDtypeStruct((E, N), jnp.int32),
            jax.ShapeDtypeStruct((E, 1), jnp.int32),
        ),
        scratch_shapes=[pltpu.VMEM((E, 1), jnp.float32)],
    )(x, Wg, bg.reshape(E, 1))


def _expert_kernel(cnt_ref, einp_ref, w1_ref, b1_ref, w2_ref, b2_ref,
                   wT_ref, posT_ref, out_ref, xg_ref, o_ref):
    e = pl.program_id(0)
    k = pl.program_id(1)

    @pl.when((e == 0) & (k == 0))
    def _init():
        out_ref[...] = jnp.zeros((N, DIM), jnp.float32)

    nblk = lax.div(cnt_ref[e] + BTC - 1, BTC)
    pos = posT_ref[pl.ds(e, 1), :]
    w = wT_ref[pl.ds(e, 1), :]
    act = w > 0.0

    @pl.when(k == 0)
    def _gather():
        def gbody(j, c):
            rr = lax.broadcasted_iota(jnp.int32, (BTC, N), 0) + j * BTC
            pmat = ((pos == rr) & act).astype(jnp.float32)
            xg_ref[pl.ds(j * BTC, BTC), :] = lax.dot_general(
                pmat, einp_ref[0], (((1,), (0,)), ((), ())),
                preferred_element_type=jnp.float32)
            return c

        lax.fori_loop(0, nblk, gbody, 0)

    def fbody(j, c):
        rows = pl.ds(j * BTC, BTC)
        h = lax.dot_general(xg_ref[rows, :], w1_ref[0],
                            (((1,), (0,)), ((), ())),
                            preferred_element_type=jnp.float32)
        h = jax.nn.gelu(h + b1_ref[0, 0])
        contrib = lax.dot_general(h, w2_ref[0], (((1,), (0,)), ((), ())),
                                  preferred_element_type=jnp.float32)
        prev = jnp.where(k == 0, 0.0, o_ref[rows, :])
        total = prev + contrib
        o_ref[rows, :] = total

        @pl.when(k == NK - 1)
        def _fuse():
            rr = lax.broadcasted_iota(jnp.int32, (BTC, N), 0) + j * BTC
            gmat = jnp.where((pos == rr) & act, w, 0.0)
            opb = total + b2_ref[pl.ds(e, 1), :]
            out_ref[...] += lax.dot_general(
                gmat, opb, (((0,), (0,)), ((), ())),
                preferred_element_type=jnp.float32)

        return c

    lax.fori_loop(0, nblk, fbody, 0)


def _experts(counts, expert_inputs, W1, b1, W2, b2, wT, posT):
    grid_spec = pltpu.PrefetchScalarGridSpec(
        num_scalar_prefetch=1,
        grid=(E, NK),
        in_specs=[
            pl.BlockSpec((1, N, DIM), lambda e, k, cnt: (e, 0, 0)),
            pl.BlockSpec((1, DIM, FFB), lambda e, k, cnt: (e, 0, k)),
            pl.BlockSpec((1, 1, FFB), lambda e, k, cnt: (e, 0, k)),
            pl.BlockSpec((1, FFB, DIM), lambda e, k, cnt: (e, k, 0)),
            pl.BlockSpec((E, DIM), lambda e, k, cnt: (0, 0)),
            pl.BlockSpec((E, N), lambda e, k, cnt: (0, 0)),
            pl.BlockSpec((E, N), lambda e, k, cnt: (0, 0)),
        ],
        out_specs=pl.BlockSpec((N, DIM), lambda e, k, cnt: (0, 0)),
        scratch_shapes=[
            pltpu.VMEM((NJ * BTC, DIM), jnp.float32),
            pltpu.VMEM((NJ * BTC, DIM), jnp.float32),
        ],
    )
    return pl.pallas_call(
        _expert_kernel,
        grid_spec=grid_spec,
        out_shape=jax.ShapeDtypeStruct((N, DIM), jnp.float32),
    )(counts, expert_inputs, W1, b1.reshape(E, 1, FF), W2, b2, wT, posT)


def kernel(x, expert_inputs, Wg, bg, W1, b1, W2, b2):
    scoresT, wT, posT, counts = _gate(x, Wg, bg)
    fused = _experts(counts.reshape(E), expert_inputs, W1, b1, W2, b2,
                     wT, posT)
    return (fused, scoresT.T)
